# baseline (speedup 1.0000x reference)
import jax
import jax.numpy as jnp
from jax import lax
from jax.experimental import pallas as pl
from jax.experimental.pallas import tpu as pltpu
from jax.experimental.pallas import tpu_sc as plsc

D_MODEL = 2048
D_SPARSE = 32768
K = 64
N_TOK = 32

BF_ENC = 2048
BF_DEC = 2048

NC, NS, L = 2, 16, 16
NV = D_SPARSE // L
UNR = 8
NB = 256
NBS = 257
HIST_WORDS = L * NBS


def _encode_body(hb_ref, w_ref, eb_ref, out_ref):
    acc = lax.dot_general(hb_ref[...], w_ref[...], (((1,), (1,)), ((), ())),
                          preferred_element_type=jnp.float32)
    out_ref[...] = acc + eb_ref[...]


def _encode(hb, W_enc, enc_bias):
    return pl.pallas_call(
        _encode_body,
        grid=(D_SPARSE // BF_ENC,),
        in_specs=[
            pl.BlockSpec((N_TOK, D_MODEL), lambda j: (0, 0)),
            pl.BlockSpec((BF_ENC, D_MODEL), lambda j: (j, 0)),
            pl.BlockSpec((1, BF_ENC), lambda j: (0, j)),
        ],
        out_specs=pl.BlockSpec((N_TOK, BF_ENC), lambda j: (0, j)),
        out_shape=jax.ShapeDtypeStruct((N_TOK, D_SPARSE), jnp.float32),
    )(hb, W_enc, enc_bias.reshape(1, -1))


def _decode_body(acts_ref, w_ref, pb_ref, out_ref, acc_ref):
    j = pl.program_id(0)

    @pl.when(j == 0)
    def _():
        acc_ref[...] = jnp.zeros_like(acc_ref)

    acc_ref[...] += lax.dot_general(acts_ref[...], w_ref[...],
                                    (((1,), (1,)), ((), ())),
                                    preferred_element_type=jnp.float32)

    @pl.when(j == pl.num_programs(0) - 1)
    def _():
        out_ref[...] = acc_ref[...] + pb_ref[...]


def _decode(acts, W_dec, pre_bias):
    return pl.pallas_call(
        _decode_body,
        grid=(D_SPARSE // BF_DEC,),
        in_specs=[
            pl.BlockSpec((N_TOK, BF_DEC), lambda j: (0, j)),
            pl.BlockSpec((D_MODEL, BF_DEC), lambda j: (0, j)),
            pl.BlockSpec((1, D_MODEL), lambda j: (0, 0)),
        ],
        out_specs=pl.BlockSpec((N_TOK, D_MODEL), lambda j: (0, 0)),
        out_shape=jax.ShapeDtypeStruct((N_TOK, D_MODEL), jnp.float32),
        scratch_shapes=[pltpu.VMEM((N_TOK, D_MODEL), jnp.float32)],
    )(acts, W_dec, pre_bias.reshape(1, -1))


def _topk_sc_body(pre_hbm, acts_hbm, fbuf, keys, hist, btot, cand, smem):
    row = lax.axis_index("s") * NC + lax.axis_index("c")
    pltpu.sync_copy(pre_hbm.at[row], fbuf)

    iota = lax.iota(jnp.int32, L)
    iota_nbs = iota * NBS
    ones = jnp.ones((L,), jnp.int32)
    zeros_i = jnp.zeros((L,), jnp.int32)
    zeros_f = jnp.zeros((L,), jnp.float32)

    def zero_hist():
        def zb(k, _):
            hist[pl.ds(k * L, L)] = zeros_i
            return 0
        lax.fori_loop(0, (HIST_WORDS + L - 1) // L, zb, 0)

    def compute_btot():
        def bt(g, _):
            acc = zeros_i
            for r in range(L):
                acc = acc + hist[pl.ds(r * NBS + g * L, L)]
                hist[pl.ds(r * NBS + g * L, L)] = zeros_i
            btot[pl.ds(g * L, L)] = acc
            return 0
        lax.fori_loop(0, NB // L, bt, 0)

    def search_desc(m):
        def body(gr, carry):
            tot, b_s, ab_s, cnt_s = carry
            g = (NB // L - 1) - gr
            tv = btot[pl.ds(g * L, L)]
            s = lax.rev(plsc.cumsum(lax.rev(tv, (0,))), (0,)) + tot
            above = s - tv
            hit = (above < m) & (s >= m)
            b_hit = jnp.max(jnp.where(hit, g * L + iota, -1))
            a_hit = jnp.max(jnp.where(hit, above, -1))
            c_hit = jnp.max(jnp.where(hit, tv, -1))
            return (tot + jnp.sum(tv), jnp.maximum(b_s, b_hit),
                    jnp.maximum(ab_s, a_hit), jnp.maximum(cnt_s, c_hit))
        init = (jnp.int32(0), jnp.int32(-1), jnp.int32(-1), jnp.int32(-1))
        _, b_s, ab_s, cnt_s = lax.fori_loop(0, NB // L, body, init)
        return b_s, ab_s, cnt_s

    def search_asc(m):
        def body(g, carry):
            tot, b_s, bel_s = carry
            tv = btot[pl.ds(g * L, L)]
            s = plsc.cumsum(tv) + tot
            below = s - tv
            hit = (below < m) & (s >= m)
            b_hit = jnp.max(jnp.where(hit, g * L + iota, -1))
            bel_hit = jnp.max(jnp.where(hit, below, -1))
            return (tot + jnp.sum(tv), jnp.maximum(b_s, b_hit),
                    jnp.maximum(bel_s, bel_hit))
        init = (jnp.int32(0), jnp.int32(-1), jnp.int32(-1))
        _, b_s, bel_s = lax.fori_loop(0, NB // L, body, init)
        return b_s, bel_s

    neg_inf = jnp.full((L,), jnp.int32(-0x80000000))

    def pa(k, carry):
        r0, r1, r2, r3 = carry
        for u in range(UNR):
            i = k * UNR + u
            v = fbuf[pl.ds(i * L, L)]
            b = lax.bitcast_convert_type(v, jnp.int32)
            key = b ^ ((b >> 31) & jnp.int32(0x7FFFFFFF))
            keys[pl.ds(i * L, L)] = key
            fbuf[pl.ds(i * L, L)] = zeros_f
            x1 = jnp.minimum(r0, key)
            r0 = jnp.maximum(r0, key)
            x2 = jnp.minimum(r1, x1)
            r1 = jnp.maximum(r1, x1)
            x3 = jnp.minimum(r2, x2)
            r2 = jnp.maximum(r2, x2)
            r3 = jnp.maximum(r3, x3)
        return r0, r1, r2, r3
    _, _, _, r3 = lax.fori_loop(0, NV // UNR, pa,
                                (neg_inf, neg_inf, neg_inf, neg_inf))
    t0 = jnp.min(r3)

    def pb_(k, off):
        for u in range(UNR):
            i = k * UNR + u
            key = keys[pl.ds(i * L, L)]
            msk = key >= t0
            posv = i * L + iota
            plsc.store_compressed(cand.at[pl.ds(off, L)], posv, mask=msk)
            off = off + plsc.all_reduce_population_count(msk)[0]
        return off
    n_cand = lax.fori_loop(0, NV // UNR, pb_, jnp.int32(0))
    nk = (n_cand + L - 1) // L

    zero_hist()

    def cand_l1(k, _):
        posv = cand[pl.ds(k * L, L)] & jnp.int32(D_SPARSE - 1)
        lanev = (k * L + iota) < n_cand
        key = plsc.load_gather(keys, [posv])
        d = (key >> 24) + 128
        plsc.addupdate_scatter(hist, [iota_nbs + d], ones, mask=lanev)
        return 0
    lax.fori_loop(0, nk, cand_l1, 0)
    compute_btot()
    b1, ab1, _ = search_desc(jnp.int32(K))
    m1 = K - ab1
    t8 = b1 - 128

    def cand_hist(mshift, mval, dshift):
        def cp(k, _):
            posv = cand[pl.ds(k * L, L)] & jnp.int32(D_SPARSE - 1)
            lanev = (k * L + iota) < n_cand
            key = plsc.load_gather(keys, [posv])
            msk = lanev & ((key >> mshift) == mval)
            d = (key >> dshift) & 0xFF
            plsc.addupdate_scatter(hist, [iota_nbs + d], ones, mask=msk)
            return 0
        lax.fori_loop(0, nk, cp, 0)

    cand_hist(24, t8, 16)
    compute_btot()
    b2, ab2, _ = search_desc(m1)
    m2 = m1 - ab2
    t16 = t8 * 256 + b2

    cand_hist(16, t16, 8)
    compute_btot()
    b3, ab3, _ = search_desc(m2)
    m3 = m2 - ab3
    t24 = t16 * 256 + b3

    cand_hist(8, t24, 0)
    compute_btot()
    b4, ab4, c_eq = search_desc(m3)
    m4 = m3 - ab4
    t32 = t24 * 256 + b4

    smem[0] = jnp.int32(D_SPARSE - 1)

    @pl.when((m4 < c_eq) & (t32 > 0))
    def _():
        def ta(k, _):
            posv = cand[pl.ds(k * L, L)] & jnp.int32(D_SPARSE - 1)
            lanev = (k * L + iota) < n_cand
            key = plsc.load_gather(keys, [posv])
            msk = lanev & (key == t32)
            plsc.addupdate_scatter(hist, [iota_nbs + (posv >> 8)], ones,
                                   mask=msk)
            return 0
        lax.fori_loop(0, nk, ta, 0)
        compute_btot()
        bpA, belA = search_asc(m4)
        mA = m4 - belA

        def tb(k, _):
            posv = cand[pl.ds(k * L, L)] & jnp.int32(D_SPARSE - 1)
            lanev = (k * L + iota) < n_cand
            key = plsc.load_gather(keys, [posv])
            msk = lanev & (key == t32) & ((posv >> 8) == bpA)
            plsc.addupdate_scatter(hist, [iota_nbs + (posv & 0xFF)], ones,
                                   mask=msk)
            return 0
        lax.fori_loop(0, nk, tb, 0)
        compute_btot()
        bpB, _ = search_asc(mA)
        smem[0] = bpA * 256 + bpB

    p_cut = smem[0]

    def fx(k, _):
        posv = cand[pl.ds(k * L, L)] & jnp.int32(D_SPARSE - 1)
        lanev = (k * L + iota) < n_cand
        key = plsc.load_gather(keys, [posv])
        sel = (key > t32) | ((key == t32) & (posv <= p_cut))
        msk = lanev & sel & (key > 0)
        val = lax.bitcast_convert_type(key, jnp.float32)
        plsc.store_scatter(fbuf, [posv], val, mask=msk)
        return 0
    lax.fori_loop(0, nk, fx, 0)

    pltpu.sync_copy(fbuf, acts_hbm.at[row])


def _topk_sc(pre):
    mesh = plsc.VectorSubcoreMesh(core_axis_name="c", subcore_axis_name="s",
                                  num_cores=NC, num_subcores=NS)
    return pl.kernel(
        _topk_sc_body,
        out_type=jax.ShapeDtypeStruct((N_TOK, D_SPARSE), jnp.float32),
        mesh=mesh,
        compiler_params=pltpu.CompilerParams(needs_layout_passes=False),
        scratch_types=[
            pltpu.VMEM((D_SPARSE,), jnp.float32),
            pltpu.VMEM((D_SPARSE,), jnp.int32),
            pltpu.VMEM((HIST_WORDS,), jnp.int32),
            pltpu.VMEM((NB,), jnp.int32),
            pltpu.VMEM((D_SPARSE + L,), jnp.int32),
            pltpu.SMEM((8,), jnp.int32),
        ],
    )(pre)


def kernel(h, W_enc, W_dec, pre_bias, enc_bias):
    hb = h - pre_bias
    pre = _encode(hb, W_enc, enc_bias)
    acts = _topk_sc(pre)
    recon = _decode(acts, W_dec, pre_bias)
    return (acts, recon)


# speedup vs baseline: 1.0538x; 1.0538x over previous
import jax
import jax.numpy as jnp
from jax import lax
from jax.experimental import pallas as pl
from jax.experimental.pallas import tpu as pltpu
from jax.experimental.pallas import tpu_sc as plsc

D_MODEL = 2048
D_SPARSE = 32768
K = 64
N_TOK = 32

BF_ENC = 2048
BF_DEC = 2048

NC, NS, L = 2, 16, 16
NV = D_SPARSE // L
UNR = 8
NB = 256
NBS = 257
HIST_WORDS = L * NBS


def _encode_body(hb_ref, w_ref, eb_ref, out_ref):
    acc = lax.dot_general(hb_ref[...], w_ref[...], (((1,), (1,)), ((), ())),
                          preferred_element_type=jnp.float32)
    out_ref[...] = acc + eb_ref[...]


def _encode(hb, W_enc, enc_bias):
    return pl.pallas_call(
        _encode_body,
        grid=(D_SPARSE // BF_ENC,),
        in_specs=[
            pl.BlockSpec((N_TOK, D_MODEL), lambda j: (0, 0)),
            pl.BlockSpec((BF_ENC, D_MODEL), lambda j: (j, 0)),
            pl.BlockSpec((1, BF_ENC), lambda j: (0, j)),
        ],
        out_specs=pl.BlockSpec((N_TOK, BF_ENC), lambda j: (0, j)),
        out_shape=jax.ShapeDtypeStruct((N_TOK, D_SPARSE), jnp.float32),
    )(hb, W_enc, enc_bias.reshape(1, -1))


def _decode_body(acts_ref, w_ref, pb_ref, out_ref, acc_ref):
    j = pl.program_id(0)

    @pl.when(j == 0)
    def _():
        acc_ref[...] = jnp.zeros_like(acc_ref)

    acc_ref[...] += lax.dot_general(acts_ref[...], w_ref[...],
                                    (((1,), (1,)), ((), ())),
                                    preferred_element_type=jnp.float32)

    @pl.when(j == pl.num_programs(0) - 1)
    def _():
        out_ref[...] = acc_ref[...] + pb_ref[...]


def _decode(acts, W_dec, pre_bias):
    return pl.pallas_call(
        _decode_body,
        grid=(D_SPARSE // BF_DEC,),
        in_specs=[
            pl.BlockSpec((N_TOK, BF_DEC), lambda j: (0, j)),
            pl.BlockSpec((D_MODEL, BF_DEC), lambda j: (0, j)),
            pl.BlockSpec((1, D_MODEL), lambda j: (0, 0)),
        ],
        out_specs=pl.BlockSpec((N_TOK, D_MODEL), lambda j: (0, 0)),
        out_shape=jax.ShapeDtypeStruct((N_TOK, D_MODEL), jnp.float32),
        scratch_shapes=[pltpu.VMEM((N_TOK, D_MODEL), jnp.float32)],
    )(acts, W_dec, pre_bias.reshape(1, -1))


def _topk_sc_body(pre_hbm, acts_hbm, fbuf, keys, hist, btot, cand, smem):
    row = lax.axis_index("s") * NC + lax.axis_index("c")
    pltpu.sync_copy(pre_hbm.at[row], fbuf)

    iota = lax.iota(jnp.int32, L)
    iota_nbs = iota * NBS
    ones = jnp.ones((L,), jnp.int32)
    zeros_i = jnp.zeros((L,), jnp.int32)
    zeros_f = jnp.zeros((L,), jnp.float32)

    def zero_hist():
        def zb(k, _):
            hist[pl.ds(k * L, L)] = zeros_i
            return 0
        lax.fori_loop(0, (HIST_WORDS + L - 1) // L, zb, 0)

    def compute_btot():
        def bt(g, _):
            acc = zeros_i
            for r in range(L):
                acc = acc + hist[pl.ds(r * NBS + g * L, L)]
                hist[pl.ds(r * NBS + g * L, L)] = zeros_i
            btot[pl.ds(g * L, L)] = acc
            return 0
        lax.fori_loop(0, NB // L, bt, 0)

    def search_desc(m):
        def body(gr, carry):
            tot, b_s, ab_s, cnt_s = carry
            g = (NB // L - 1) - gr
            tv = btot[pl.ds(g * L, L)]
            s = lax.rev(plsc.cumsum(lax.rev(tv, (0,))), (0,)) + tot
            above = s - tv
            hit = (above < m) & (s >= m)
            b_hit = jnp.max(jnp.where(hit, g * L + iota, -1))
            a_hit = jnp.max(jnp.where(hit, above, -1))
            c_hit = jnp.max(jnp.where(hit, tv, -1))
            return (tot + jnp.sum(tv), jnp.maximum(b_s, b_hit),
                    jnp.maximum(ab_s, a_hit), jnp.maximum(cnt_s, c_hit))
        init = (jnp.int32(0), jnp.int32(-1), jnp.int32(-1), jnp.int32(-1))
        _, b_s, ab_s, cnt_s = lax.fori_loop(0, NB // L, body, init)
        return b_s, ab_s, cnt_s

    def search_asc(m):
        def body(g, carry):
            tot, b_s, bel_s = carry
            tv = btot[pl.ds(g * L, L)]
            s = plsc.cumsum(tv) + tot
            below = s - tv
            hit = (below < m) & (s >= m)
            b_hit = jnp.max(jnp.where(hit, g * L + iota, -1))
            bel_hit = jnp.max(jnp.where(hit, below, -1))
            return (tot + jnp.sum(tv), jnp.maximum(b_s, b_hit),
                    jnp.maximum(bel_s, bel_hit))
        init = (jnp.int32(0), jnp.int32(-1), jnp.int32(-1))
        _, b_s, bel_s = lax.fori_loop(0, NB // L, body, init)
        return b_s, bel_s

    neg_inf = jnp.full((L,), jnp.int32(-0x80000000))

    def pa(k, carry):
        r0, r1, r2, r3 = carry
        for u in range(UNR):
            i = k * UNR + u
            v = fbuf[pl.ds(i * L, L)]
            b = lax.bitcast_convert_type(v, jnp.int32)
            key = b ^ ((b >> 31) & jnp.int32(0x7FFFFFFF))
            keys[pl.ds(i * L, L)] = key
            fbuf[pl.ds(i * L, L)] = zeros_f
            x1 = jnp.minimum(r0, key)
            r0 = jnp.maximum(r0, key)
            x2 = jnp.minimum(r1, x1)
            r1 = jnp.maximum(r1, x1)
            x3 = jnp.minimum(r2, x2)
            r2 = jnp.maximum(r2, x2)
            r3 = jnp.maximum(r3, x3)
        return r0, r1, r2, r3
    _, _, _, r3 = lax.fori_loop(0, NV // UNR, pa,
                                (neg_inf, neg_inf, neg_inf, neg_inf))
    t0 = jnp.min(r3)

    def pb_(k, off):
        msks = []
        tot = jnp.int32(0)
        for u in range(UNR):
            i = k * UNR + u
            key = keys[pl.ds(i * L, L)]
            msk = key >= t0
            msks.append(msk)
            tot = tot + plsc.all_reduce_population_count(msk)[0]

        @pl.when(tot > 0)
        def _():
            o = off
            for u in range(UNR):
                i = k * UNR + u
                posv = i * L + iota
                plsc.store_compressed(cand.at[pl.ds(o, L)], posv, mask=msks[u])
                o = o + plsc.all_reduce_population_count(msks[u])[0]
        return off + tot
    n_cand = lax.fori_loop(0, NV // UNR, pb_, jnp.int32(0))
    nk = (n_cand + L - 1) // L

    zero_hist()

    def cand_l1(k, _):
        posv = cand[pl.ds(k * L, L)] & jnp.int32(D_SPARSE - 1)
        lanev = (k * L + iota) < n_cand
        key = plsc.load_gather(keys, [posv])
        d = (key >> 24) + 128
        plsc.addupdate_scatter(hist, [iota_nbs + d], ones, mask=lanev)
        return 0
    lax.fori_loop(0, nk, cand_l1, 0)
    compute_btot()
    b1, ab1, _ = search_desc(jnp.int32(K))
    m1 = K - ab1
    t8 = b1 - 128

    def cand_hist(mshift, mval, dshift):
        def cp(k, _):
            posv = cand[pl.ds(k * L, L)] & jnp.int32(D_SPARSE - 1)
            lanev = (k * L + iota) < n_cand
            key = plsc.load_gather(keys, [posv])
            msk = lanev & ((key >> mshift) == mval)
            d = (key >> dshift) & 0xFF
            plsc.addupdate_scatter(hist, [iota_nbs + d], ones, mask=msk)
            return 0
        lax.fori_loop(0, nk, cp, 0)

    cand_hist(24, t8, 16)
    compute_btot()
    b2, ab2, _ = search_desc(m1)
    m2 = m1 - ab2
    t16 = t8 * 256 + b2

    cand_hist(16, t16, 8)
    compute_btot()
    b3, ab3, _ = search_desc(m2)
    m3 = m2 - ab3
    t24 = t16 * 256 + b3

    cand_hist(8, t24, 0)
    compute_btot()
    b4, ab4, c_eq = search_desc(m3)
    m4 = m3 - ab4
    t32 = t24 * 256 + b4

    smem[0] = jnp.int32(D_SPARSE - 1)

    @pl.when((m4 < c_eq) & (t32 > 0))
    def _():
        def ta(k, _):
            posv = cand[pl.ds(k * L, L)] & jnp.int32(D_SPARSE - 1)
            lanev = (k * L + iota) < n_cand
            key = plsc.load_gather(keys, [posv])
            msk = lanev & (key == t32)
            plsc.addupdate_scatter(hist, [iota_nbs + (posv >> 8)], ones,
                                   mask=msk)
            return 0
        lax.fori_loop(0, nk, ta, 0)
        compute_btot()
        bpA, belA = search_asc(m4)
        mA = m4 - belA

        def tb(k, _):
            posv = cand[pl.ds(k * L, L)] & jnp.int32(D_SPARSE - 1)
            lanev = (k * L + iota) < n_cand
            key = plsc.load_gather(keys, [posv])
            msk = lanev & (key == t32) & ((posv >> 8) == bpA)
            plsc.addupdate_scatter(hist, [iota_nbs + (posv & 0xFF)], ones,
                                   mask=msk)
            return 0
        lax.fori_loop(0, nk, tb, 0)
        compute_btot()
        bpB, _ = search_asc(mA)
        smem[0] = bpA * 256 + bpB

    p_cut = smem[0]

    def fx(k, _):
        posv = cand[pl.ds(k * L, L)] & jnp.int32(D_SPARSE - 1)
        lanev = (k * L + iota) < n_cand
        key = plsc.load_gather(keys, [posv])
        sel = (key > t32) | ((key == t32) & (posv <= p_cut))
        msk = lanev & sel & (key > 0)
        val = lax.bitcast_convert_type(key, jnp.float32)
        plsc.store_scatter(fbuf, [posv], val, mask=msk)
        return 0
    lax.fori_loop(0, nk, fx, 0)

    pltpu.sync_copy(fbuf, acts_hbm.at[row])


def _topk_sc(pre):
    mesh = plsc.VectorSubcoreMesh(core_axis_name="c", subcore_axis_name="s",
                                  num_cores=NC, num_subcores=NS)
    return pl.kernel(
        _topk_sc_body,
        out_type=jax.ShapeDtypeStruct((N_TOK, D_SPARSE), jnp.float32),
        mesh=mesh,
        compiler_params=pltpu.CompilerParams(needs_layout_passes=False),
        scratch_types=[
            pltpu.VMEM((D_SPARSE,), jnp.float32),
            pltpu.VMEM((D_SPARSE,), jnp.int32),
            pltpu.VMEM((HIST_WORDS,), jnp.int32),
            pltpu.VMEM((NB,), jnp.int32),
            pltpu.VMEM((D_SPARSE + L,), jnp.int32),
            pltpu.SMEM((8,), jnp.int32),
        ],
    )(pre)


def kernel(h, W_enc, W_dec, pre_bias, enc_bias):
    hb = h - pre_bias
    pre = _encode(hb, W_enc, enc_bias)
    acts = _topk_sc(pre)
    recon = _decode(acts, W_dec, pre_bias)
    return (acts, recon)
